# initial kernel scaffold (unmeasured)
import jax
import jax.numpy as jnp
from jax import lax
from jax.experimental import pallas as pl
from jax.experimental.pallas import tpu as pltpu

N_Z = 4
T = 2048
D = 1024
CHUNK = T // N_Z
N_HOPS = 2 * (N_Z - 1)


def _allreduce_z(partial):

    def body(p_ref, out_ref, comm_ref, send_sems, recv_sems):
        my_x = lax.axis_index("x")
        my_y = lax.axis_index("y")
        my_z = lax.axis_index("z")
        right = (my_x, my_y, (my_z + 1) % N_Z)
        left = (my_x, my_y, (my_z - 1) % N_Z)

        barrier_sem = pltpu.get_barrier_semaphore()
        for nbr in (left, right):
            pl.semaphore_signal(
                barrier_sem,
                inc=1,
                device_id=nbr,
                device_id_type=pl.DeviceIdType.MESH,
            )
        pl.semaphore_wait(barrier_sem, 2)

        out_ref[:, :] = p_ref[:, :]

        for h in range(N_Z - 1):
            send_idx = (my_z - h) % N_Z
            recv_idx = (my_z - h - 1) % N_Z
            rdma = pltpu.make_async_remote_copy(
                src_ref=out_ref.at[pl.ds(send_idx * CHUNK, CHUNK)],
                dst_ref=comm_ref.at[h],
                send_sem=send_sems.at[h],
                recv_sem=recv_sems.at[h],
                device_id=right,
                device_id_type=pl.DeviceIdType.MESH,
            )
            rdma.start()
            rdma.wait()
            out_ref[pl.ds(recv_idx * CHUNK, CHUNK), :] = (
                out_ref[pl.ds(recv_idx * CHUNK, CHUNK), :] + comm_ref[h, :, :]
            )

        for g in range(N_Z - 1):
            h = (N_Z - 1) + g
            send_idx = (my_z + 1 - g) % N_Z
            recv_idx = (my_z - g) % N_Z
            rdma = pltpu.make_async_remote_copy(
                src_ref=out_ref.at[pl.ds(send_idx * CHUNK, CHUNK)],
                dst_ref=comm_ref.at[h],
                send_sem=send_sems.at[h],
                recv_sem=recv_sems.at[h],
                device_id=right,
                device_id_type=pl.DeviceIdType.MESH,
            )
            rdma.start()
            rdma.wait()
            out_ref[pl.ds(recv_idx * CHUNK, CHUNK), :] = comm_ref[h, :, :]

    return pl.pallas_call(
        body,
        out_shape=jax.ShapeDtypeStruct((T, D), jnp.float32),
        in_specs=[pl.BlockSpec(memory_space=pltpu.VMEM)],
        out_specs=pl.BlockSpec(memory_space=pltpu.VMEM),
        scratch_shapes=[
            pltpu.VMEM((N_HOPS, CHUNK, D), jnp.float32),
            pltpu.SemaphoreType.DMA((N_HOPS,)),
            pltpu.SemaphoreType.DMA((N_HOPS,)),
        ],
        compiler_params=pltpu.CompilerParams(collective_id=0),
    )(partial)


def kernel(ids, E):
    v_local = E.shape[0]
    my_z = lax.axis_index("z")
    local = ids - my_z * v_local
    in_range = (local >= 0) & (local < v_local)
    safe = jnp.where(in_range, local, 0)
    partial = jnp.take(E, safe, axis=0) * in_range[:, None].astype(E.dtype)
    return _allreduce_z(partial)


# baseline (device time: 217095 ns/iter reference)
import jax
import jax.numpy as jnp
from jax import lax
from jax.experimental import pallas as pl
from jax.experimental.pallas import tpu as pltpu

N_Z = 4
T = 2048
D = 1024
CHUNK = T // N_Z
N_HOPS = 2 * (N_Z - 1)


def _vembed_allreduce(safe_ids, mask, E):
    def body(ids_ref, mask_ref, e_ref, out_ref, gat_ref, comm_ref,
             row_sem, send_sems, recv_sems):
        my_x = lax.axis_index("x")
        my_y = lax.axis_index("y")
        my_z = lax.axis_index("z")
        right = (my_x, my_y, (my_z + 1) % N_Z)
        left = (my_x, my_y, (my_z - 1) % N_Z)

        def issue(i, c):
            vid = ids_ref[i]
            pltpu.make_async_copy(
                e_ref.at[pl.ds(vid, 1)], gat_ref.at[pl.ds(i, 1)], row_sem
            ).start()
            return c

        lax.fori_loop(0, T, issue, 0)

        barrier_sem = pltpu.get_barrier_semaphore()
        for nbr in (left, right):
            pl.semaphore_signal(
                barrier_sem,
                inc=1,
                device_id=nbr,
                device_id_type=pl.DeviceIdType.MESH,
            )
        pl.semaphore_wait(barrier_sem, 2)

        def drain(i, c):
            pltpu.make_async_copy(
                e_ref.at[pl.ds(0, 1)], gat_ref.at[pl.ds(0, 1)], row_sem
            ).wait()
            return c

        lax.fori_loop(0, T, drain, 0)

        out_ref[:, :] = jnp.where(mask_ref[:, :] != 0, gat_ref[:, :], 0.0)

        for h in range(N_Z - 1):
            send_idx = (my_z - h) % N_Z
            recv_idx = (my_z - h - 1) % N_Z
            rdma = pltpu.make_async_remote_copy(
                src_ref=out_ref.at[pl.ds(send_idx * CHUNK, CHUNK)],
                dst_ref=comm_ref.at[h],
                send_sem=send_sems.at[h],
                recv_sem=recv_sems.at[h],
                device_id=right,
                device_id_type=pl.DeviceIdType.MESH,
            )
            rdma.start()
            rdma.wait()
            out_ref[pl.ds(recv_idx * CHUNK, CHUNK), :] = (
                out_ref[pl.ds(recv_idx * CHUNK, CHUNK), :] + comm_ref[h, :, :]
            )

        for g in range(N_Z - 1):
            h = (N_Z - 1) + g
            send_idx = (my_z + 1 - g) % N_Z
            recv_idx = (my_z - g) % N_Z
            rdma = pltpu.make_async_remote_copy(
                src_ref=out_ref.at[pl.ds(send_idx * CHUNK, CHUNK)],
                dst_ref=comm_ref.at[h],
                send_sem=send_sems.at[h],
                recv_sem=recv_sems.at[h],
                device_id=right,
                device_id_type=pl.DeviceIdType.MESH,
            )
            rdma.start()
            rdma.wait()
            out_ref[pl.ds(recv_idx * CHUNK, CHUNK), :] = comm_ref[h, :, :]

    return pl.pallas_call(
        body,
        out_shape=jax.ShapeDtypeStruct((T, D), jnp.float32),
        in_specs=[
            pl.BlockSpec(memory_space=pltpu.SMEM),
            pl.BlockSpec(memory_space=pltpu.VMEM),
            pl.BlockSpec(memory_space=pl.ANY),
        ],
        out_specs=pl.BlockSpec(memory_space=pltpu.VMEM),
        scratch_shapes=[
            pltpu.VMEM((T, D), jnp.float32),
            pltpu.VMEM((N_HOPS, CHUNK, D), jnp.float32),
            pltpu.SemaphoreType.DMA,
            pltpu.SemaphoreType.DMA((N_HOPS,)),
            pltpu.SemaphoreType.DMA((N_HOPS,)),
        ],
        compiler_params=pltpu.CompilerParams(collective_id=0),
    )(safe_ids, mask, E)


def kernel(ids, E):
    v_local = E.shape[0]
    my_z = lax.axis_index("z")
    local = ids - my_z * v_local
    in_range = (local >= 0) & (local < v_local)
    safe_ids = jnp.where(in_range, local, 0)
    mask = in_range[:, None].astype(jnp.float32)
    return _vembed_allreduce(safe_ids, mask, E)


# device time: 203052 ns/iter; 1.0692x vs baseline; 1.0692x over previous
import jax
import jax.numpy as jnp
from jax import lax
from jax.experimental import pallas as pl
from jax.experimental.pallas import tpu as pltpu

N_Z = 4
T = 2048
D = 1024
HALF = T // 2
CH = HALF // N_Z
N_HOPS = 2 * (N_Z - 1)


def _vembed_allreduce(safe_ids, mask, E):
    def body(ids_ref, mask_ref, e_ref, out_ref, gat_ref, comm_a, comm_b,
             gsems, send_a, recv_a, send_b, recv_b):
        my_x = lax.axis_index("x")
        my_y = lax.axis_index("y")
        my_z = lax.axis_index("z")
        right = (my_x, my_y, (my_z + 1) % N_Z)
        left = (my_x, my_y, (my_z - 1) % N_Z)

        def a_base(k):
            return ((my_z - k) % N_Z) * CH

        def b_base(k):
            return HALF + ((my_z + k) % N_Z) * CH

        for k in range(N_Z):
            base_a = a_base(k)
            base_b = b_base(k)

            def issue(i, c, base_a=base_a, base_b=base_b, k=k):
                pltpu.make_async_copy(
                    e_ref.at[pl.ds(ids_ref[base_a + i], 1)],
                    gat_ref.at[pl.ds(base_a + i, 1)],
                    gsems.at[k],
                ).start()
                pltpu.make_async_copy(
                    e_ref.at[pl.ds(ids_ref[base_b + i], 1)],
                    gat_ref.at[pl.ds(base_b + i, 1)],
                    gsems.at[k],
                ).start()
                return c

            lax.fori_loop(0, CH, issue, 0)

        barrier_sem = pltpu.get_barrier_semaphore()
        for nbr in (left, right):
            pl.semaphore_signal(
                barrier_sem,
                inc=1,
                device_id=nbr,
                device_id_type=pl.DeviceIdType.MESH,
            )
        pl.semaphore_wait(barrier_sem, 2)

        def mask_store(base):
            out_ref[pl.ds(base, CH), :] = jnp.where(
                mask_ref[pl.ds(base, CH), :] != 0,
                gat_ref[pl.ds(base, CH), :],
                0.0,
            )

        def ready_group(k):
            def w(i, c):
                pltpu.make_async_copy(
                    e_ref.at[pl.ds(0, 1)], gat_ref.at[pl.ds(0, 1)], gsems.at[k]
                ).wait()
                return c

            lax.fori_loop(0, 2 * CH, w, 0)
            mask_store(a_base(k))
            mask_store(b_base(k))

        def hop_rdmas(h, src_a, src_b):
            rdma_a = pltpu.make_async_remote_copy(
                src_ref=src_a,
                dst_ref=comm_a.at[h],
                send_sem=send_a.at[h],
                recv_sem=recv_a.at[h],
                device_id=right,
                device_id_type=pl.DeviceIdType.MESH,
            )
            rdma_b = pltpu.make_async_remote_copy(
                src_ref=src_b,
                dst_ref=comm_b.at[h],
                send_sem=send_b.at[h],
                recv_sem=recv_b.at[h],
                device_id=left,
                device_id_type=pl.DeviceIdType.MESH,
            )
            return rdma_a, rdma_b

        ready_group(0)

        for h in range(N_Z - 1):
            rdma_a, rdma_b = hop_rdmas(
                h,
                out_ref.at[pl.ds(a_base(h), CH)],
                out_ref.at[pl.ds(b_base(h), CH)],
            )
            rdma_a.start()
            rdma_b.start()
            ready_group(h + 1)
            rdma_a.wait()
            rdma_b.wait()
            ra = a_base(h + 1)
            rb = b_base(h + 1)
            out_ref[pl.ds(ra, CH), :] = (
                out_ref[pl.ds(ra, CH), :] + comm_a[h, :, :]
            )
            out_ref[pl.ds(rb, CH), :] = (
                out_ref[pl.ds(rb, CH), :] + comm_b[h, :, :]
            )

        for g in range(N_Z - 1):
            h = (N_Z - 1) + g
            if g == 0:
                src_a = out_ref.at[pl.ds(a_base(-1), CH)]
                src_b = out_ref.at[pl.ds(b_base(-1), CH)]
            else:
                src_a = comm_a.at[h - 1]
                src_b = comm_b.at[h - 1]
            rdma_a, rdma_b = hop_rdmas(h, src_a, src_b)
            rdma_a.start()
            rdma_b.start()
            rdma_a.wait()
            rdma_b.wait()
            out_ref[pl.ds(a_base(g), CH), :] = comm_a[h, :, :]
            out_ref[pl.ds(b_base(g), CH), :] = comm_b[h, :, :]

    return pl.pallas_call(
        body,
        out_shape=jax.ShapeDtypeStruct((T, D), jnp.float32),
        in_specs=[
            pl.BlockSpec(memory_space=pltpu.SMEM),
            pl.BlockSpec(memory_space=pltpu.VMEM),
            pl.BlockSpec(memory_space=pl.ANY),
        ],
        out_specs=pl.BlockSpec(memory_space=pltpu.VMEM),
        scratch_shapes=[
            pltpu.VMEM((T, D), jnp.float32),
            pltpu.VMEM((N_HOPS, CH, D), jnp.float32),
            pltpu.VMEM((N_HOPS, CH, D), jnp.float32),
            pltpu.SemaphoreType.DMA((N_Z,)),
            pltpu.SemaphoreType.DMA((N_HOPS,)),
            pltpu.SemaphoreType.DMA((N_HOPS,)),
            pltpu.SemaphoreType.DMA((N_HOPS,)),
            pltpu.SemaphoreType.DMA((N_HOPS,)),
        ],
        compiler_params=pltpu.CompilerParams(collective_id=0),
    )(safe_ids, mask, E)


def kernel(ids, E):
    v_local = E.shape[0]
    my_z = lax.axis_index("z")
    local = ids - my_z * v_local
    in_range = (local >= 0) & (local < v_local)
    safe_ids = jnp.where(in_range, local, 0)
    mask = in_range[:, None].astype(jnp.float32)
    return _vembed_allreduce(safe_ids, mask, E)


# device time: 99799 ns/iter; 2.1753x vs baseline; 2.0346x over previous
import jax
import jax.numpy as jnp
from jax import lax
from jax.experimental import pallas as pl
from jax.experimental.pallas import tpu as pltpu

N_Z = 4
N_XY = 8
T = 2048
D = 1024
P = T // N_XY
HALF = P // 2
CH = HALF // N_Z
N_HOPS = 2 * (N_Z - 1)


def _xy_coords(t):
    x = jnp.where(t < 4, 0, 1)
    y = jnp.where(t < 4, t, 7 - t)
    return x, y


def _vembed_allreduce(ids_p, mask_p, E):
    def body(ids_ref, mask_ref, e_ref, out_ref, gat_ref, red_ref,
             comm_a, comm_b, gsem,
             send_a, recv_a, send_b, recv_b,
             send_r, recv_r, send_l, recv_l):
        my_x = lax.axis_index("x")
        my_y = lax.axis_index("y")
        my_z = lax.axis_index("z")
        z_right = (my_x, my_y, (my_z + 1) % N_Z)
        z_left = (my_x, my_y, (my_z - 1) % N_Z)

        my_r = jnp.where(my_x == 0, my_y, 7 - my_y)
        rx, ry = _xy_coords((my_r + 1) % N_XY)
        lx, ly = _xy_coords((my_r + N_XY - 1) % N_XY)
        xy_right = (rx, ry, my_z)
        xy_left = (lx, ly, my_z)

        def issue(i, c):
            pltpu.make_async_copy(
                e_ref.at[pl.ds(ids_ref[i], 1)], gat_ref.at[pl.ds(i, 1)], gsem
            ).start()
            return c

        lax.fori_loop(0, P, issue, 0)

        barrier_sem = pltpu.get_barrier_semaphore()
        for nbr in (z_left, z_right, xy_left, xy_right):
            pl.semaphore_signal(
                barrier_sem,
                inc=1,
                device_id=nbr,
                device_id_type=pl.DeviceIdType.MESH,
            )
        pl.semaphore_wait(barrier_sem, 4)

        def drain(i, c):
            pltpu.make_async_copy(
                e_ref.at[pl.ds(0, 1)], gat_ref.at[pl.ds(0, 1)], gsem
            ).wait()
            return c

        lax.fori_loop(0, P, drain, 0)

        red_ref[:, :] = jnp.where(mask_ref[:, :] != 0, gat_ref[:, :], 0.0)

        def a_base(k):
            return ((my_z - k) % N_Z) * CH

        def b_base(k):
            return HALF + ((my_z + k) % N_Z) * CH

        def hop_rdmas(h, src_a, src_b):
            rdma_a = pltpu.make_async_remote_copy(
                src_ref=src_a,
                dst_ref=comm_a.at[h],
                send_sem=send_a.at[h],
                recv_sem=recv_a.at[h],
                device_id=z_right,
                device_id_type=pl.DeviceIdType.MESH,
            )
            rdma_b = pltpu.make_async_remote_copy(
                src_ref=src_b,
                dst_ref=comm_b.at[h],
                send_sem=send_b.at[h],
                recv_sem=recv_b.at[h],
                device_id=z_left,
                device_id_type=pl.DeviceIdType.MESH,
            )
            return rdma_a, rdma_b

        for h in range(N_Z - 1):
            rdma_a, rdma_b = hop_rdmas(
                h,
                red_ref.at[pl.ds(a_base(h), CH)],
                red_ref.at[pl.ds(b_base(h), CH)],
            )
            rdma_a.start()
            rdma_b.start()
            rdma_a.wait()
            rdma_b.wait()
            ra = a_base(h + 1)
            rb = b_base(h + 1)
            red_ref[pl.ds(ra, CH), :] = red_ref[pl.ds(ra, CH), :] + comm_a[h, :, :]
            red_ref[pl.ds(rb, CH), :] = red_ref[pl.ds(rb, CH), :] + comm_b[h, :, :]

        for g in range(N_Z - 1):
            h = (N_Z - 1) + g
            if g == 0:
                src_a = red_ref.at[pl.ds(a_base(-1), CH)]
                src_b = red_ref.at[pl.ds(b_base(-1), CH)]
            else:
                src_a = comm_a.at[h - 1]
                src_b = comm_b.at[h - 1]
            rdma_a, rdma_b = hop_rdmas(h, src_a, src_b)
            rdma_a.start()
            rdma_b.start()
            rdma_a.wait()
            rdma_b.wait()
            red_ref[pl.ds(a_base(g), CH), :] = comm_a[h, :, :]
            red_ref[pl.ds(b_base(g), CH), :] = comm_b[h, :, :]

        out_ref[pl.ds(my_r * P, P), :] = red_ref[:, :]

        for t in range(N_XY // 2):
            sr = ((my_r - t) % N_XY) * P
            rdma_r = pltpu.make_async_remote_copy(
                src_ref=out_ref.at[pl.ds(sr, P)],
                dst_ref=out_ref.at[pl.ds(sr, P)],
                send_sem=send_r.at[t],
                recv_sem=recv_r.at[t],
                device_id=xy_right,
                device_id_type=pl.DeviceIdType.MESH,
            )
            rdma_r.start()
            if t < N_XY // 2 - 1:
                sl = ((my_r + t) % N_XY) * P
                rdma_l = pltpu.make_async_remote_copy(
                    src_ref=out_ref.at[pl.ds(sl, P)],
                    dst_ref=out_ref.at[pl.ds(sl, P)],
                    send_sem=send_l.at[t],
                    recv_sem=recv_l.at[t],
                    device_id=xy_left,
                    device_id_type=pl.DeviceIdType.MESH,
                )
                rdma_l.start()
                rdma_l.wait()
            rdma_r.wait()

    return pl.pallas_call(
        body,
        out_shape=jax.ShapeDtypeStruct((T, D), jnp.float32),
        in_specs=[
            pl.BlockSpec(memory_space=pltpu.SMEM),
            pl.BlockSpec(memory_space=pltpu.VMEM),
            pl.BlockSpec(memory_space=pl.ANY),
        ],
        out_specs=pl.BlockSpec(memory_space=pltpu.VMEM),
        scratch_shapes=[
            pltpu.VMEM((P, D), jnp.float32),
            pltpu.VMEM((P, D), jnp.float32),
            pltpu.VMEM((N_HOPS, CH, D), jnp.float32),
            pltpu.VMEM((N_HOPS, CH, D), jnp.float32),
            pltpu.SemaphoreType.DMA,
            pltpu.SemaphoreType.DMA((N_HOPS,)),
            pltpu.SemaphoreType.DMA((N_HOPS,)),
            pltpu.SemaphoreType.DMA((N_HOPS,)),
            pltpu.SemaphoreType.DMA((N_HOPS,)),
            pltpu.SemaphoreType.DMA((N_XY // 2,)),
            pltpu.SemaphoreType.DMA((N_XY // 2,)),
            pltpu.SemaphoreType.DMA((N_XY // 2 - 1,)),
            pltpu.SemaphoreType.DMA((N_XY // 2 - 1,)),
        ],
        compiler_params=pltpu.CompilerParams(collective_id=0),
    )(ids_p, mask_p, E)


def kernel(ids, E):
    v_local = E.shape[0]
    my_x = lax.axis_index("x")
    my_y = lax.axis_index("y")
    my_z = lax.axis_index("z")
    my_r = jnp.where(my_x == 0, my_y, 7 - my_y)
    ids_p = lax.dynamic_slice_in_dim(ids, my_r * P, P)
    local = ids_p - my_z * v_local
    in_range = (local >= 0) & (local < v_local)
    safe_ids = jnp.where(in_range, local, 0)
    mask = in_range[:, None].astype(jnp.float32)
    return _vembed_allreduce(safe_ids, mask, E)


# device time: 85663 ns/iter; 2.5343x vs baseline; 1.1650x over previous
import jax
import jax.numpy as jnp
from jax import lax
from jax.experimental import pallas as pl
from jax.experimental.pallas import tpu as pltpu

N_Z = 4
N_XY = 8
T = 2048
D = 1024
P = T // N_XY
HALF = P // 2
CH = HALF // N_Z
N_HOPS = 2 * (N_Z - 1)


def _xy_coords(t):
    x = jnp.where(t < 4, 0, 1)
    y = jnp.where(t < 4, t, 7 - t)
    return x, y


def _vembed_allreduce(ids_p, mask_p, E):
    def body(ids_ref, mask_ref, e_ref, out_ref, gat_ref, red_ref,
             comm_a, comm_b, gsem,
             send_a, recv_a, send_b, recv_b,
             send_ra, recv_ra, send_rb, recv_rb,
             send_la, recv_la, send_lb, recv_lb):
        my_x = lax.axis_index("x")
        my_y = lax.axis_index("y")
        my_z = lax.axis_index("z")
        z_right = (my_x, my_y, (my_z + 1) % N_Z)
        z_left = (my_x, my_y, (my_z - 1) % N_Z)

        my_r = jnp.where(my_x == 0, my_y, 7 - my_y)
        rx, ry = _xy_coords((my_r + 1) % N_XY)
        lx, ly = _xy_coords((my_r + N_XY - 1) % N_XY)
        xy_right = (rx, ry, my_z)
        xy_left = (lx, ly, my_z)

        def issue(i, c):
            pltpu.make_async_copy(
                e_ref.at[pl.ds(ids_ref[i], 1)], gat_ref.at[pl.ds(i, 1)], gsem
            ).start()
            return c

        lax.fori_loop(0, P, issue, 0)

        barrier_sem = pltpu.get_barrier_semaphore()
        for nbr in (z_left, z_right, xy_left, xy_right):
            pl.semaphore_signal(
                barrier_sem,
                inc=1,
                device_id=nbr,
                device_id_type=pl.DeviceIdType.MESH,
            )
        pl.semaphore_wait(barrier_sem, 4)

        def drain(i, c):
            pltpu.make_async_copy(
                e_ref.at[pl.ds(0, 1)], gat_ref.at[pl.ds(0, 1)], gsem
            ).wait()
            return c

        lax.fori_loop(0, P, drain, 0)

        red_ref[:, :] = jnp.where(mask_ref[:, :] != 0, gat_ref[:, :], 0.0)

        def a_base(k):
            return ((my_z - k) % N_Z) * CH

        def b_base(k):
            return HALF + ((my_z + k) % N_Z) * CH

        def hop_rdmas(h, src_a, src_b):
            rdma_a = pltpu.make_async_remote_copy(
                src_ref=src_a,
                dst_ref=comm_a.at[h],
                send_sem=send_a.at[h],
                recv_sem=recv_a.at[h],
                device_id=z_right,
                device_id_type=pl.DeviceIdType.MESH,
            )
            rdma_b = pltpu.make_async_remote_copy(
                src_ref=src_b,
                dst_ref=comm_b.at[h],
                send_sem=send_b.at[h],
                recv_sem=recv_b.at[h],
                device_id=z_left,
                device_id_type=pl.DeviceIdType.MESH,
            )
            return rdma_a, rdma_b

        for h in range(N_Z - 1):
            rdma_a, rdma_b = hop_rdmas(
                h,
                red_ref.at[pl.ds(a_base(h), CH)],
                red_ref.at[pl.ds(b_base(h), CH)],
            )
            rdma_a.start()
            rdma_b.start()
            rdma_a.wait()
            rdma_b.wait()
            ra = a_base(h + 1)
            rb = b_base(h + 1)
            red_ref[pl.ds(ra, CH), :] = red_ref[pl.ds(ra, CH), :] + comm_a[h, :, :]
            red_ref[pl.ds(rb, CH), :] = red_ref[pl.ds(rb, CH), :] + comm_b[h, :, :]

        def lane_ab(L):
            if L == 0:
                return (my_z + 1) % N_Z, (my_z - 1) % N_Z
            return (my_z - (L - 1)) % N_Z, (my_z + (L - 1)) % N_Z

        def lane_hop_rdmas(L, t):
            aL, bL = lane_ab(L)
            i = L * 4 + t
            qr = ((my_r - t) % N_XY) * P
            ql = ((my_r + t) % N_XY) * P

            def mk(rows, send, recv, dev):
                return pltpu.make_async_remote_copy(
                    src_ref=out_ref.at[pl.ds(rows, CH)],
                    dst_ref=out_ref.at[pl.ds(rows, CH)],
                    send_sem=send.at[i],
                    recv_sem=recv.at[i],
                    device_id=dev,
                    device_id_type=pl.DeviceIdType.MESH,
                )

            ra = mk(qr + aL * CH, send_ra, recv_ra, xy_right)
            rb = mk(qr + HALF + bL * CH, send_rb, recv_rb, xy_right)
            la = mk(ql + aL * CH, send_la, recv_la, xy_left)
            lb = mk(ql + HALF + bL * CH, send_lb, recv_lb, xy_left)
            return ra, rb, la, lb

        def start_lane_hop(L, t):
            ra, rb, la, lb = lane_hop_rdmas(L, t)
            ra.start()
            rb.start()
            if t < N_XY // 2 - 1:
                la.start()
                lb.start()

        def wait_lane_hop(L, t):
            ra, rb, la, lb = lane_hop_rdmas(L, t)
            ra.wait()
            rb.wait()
            if t < N_XY // 2 - 1:
                la.wait()
                lb.wait()

        a0, b0 = lane_ab(0)
        my_base = my_r * P
        out_ref[pl.ds(my_base + a0 * CH, CH), :] = red_ref[pl.ds(a0 * CH, CH), :]
        out_ref[pl.ds(my_base + HALF + b0 * CH, CH), :] = (
            red_ref[pl.ds(HALF + b0 * CH, CH), :]
        )
        start_lane_hop(0, 0)

        for g in range(N_Z - 1):
            h = (N_Z - 1) + g
            if g == 0:
                src_a = red_ref.at[pl.ds(a_base(-1), CH)]
                src_b = red_ref.at[pl.ds(b_base(-1), CH)]
            else:
                src_a = comm_a.at[h - 1]
                src_b = comm_b.at[h - 1]
            rdma_a, rdma_b = hop_rdmas(h, src_a, src_b)
            rdma_a.start()
            rdma_b.start()
            rdma_a.wait()
            rdma_b.wait()
            out_ref[pl.ds(my_base + a_base(g), CH), :] = comm_a[h, :, :]
            out_ref[pl.ds(my_base + b_base(g), CH), :] = comm_b[h, :, :]
            start_lane_hop(g + 1, 0)

        for t in range(1, N_XY // 2):
            for L in range(N_Z):
                wait_lane_hop(L, t - 1)
                start_lane_hop(L, t)
        for L in range(N_Z):
            wait_lane_hop(L, N_XY // 2 - 1)

    return pl.pallas_call(
        body,
        out_shape=jax.ShapeDtypeStruct((T, D), jnp.float32),
        in_specs=[
            pl.BlockSpec(memory_space=pltpu.SMEM),
            pl.BlockSpec(memory_space=pltpu.VMEM),
            pl.BlockSpec(memory_space=pl.ANY),
        ],
        out_specs=pl.BlockSpec(memory_space=pltpu.VMEM),
        scratch_shapes=[
            pltpu.VMEM((P, D), jnp.float32),
            pltpu.VMEM((P, D), jnp.float32),
            pltpu.VMEM((N_HOPS, CH, D), jnp.float32),
            pltpu.VMEM((N_HOPS, CH, D), jnp.float32),
            pltpu.SemaphoreType.DMA,
            pltpu.SemaphoreType.DMA((N_HOPS,)),
            pltpu.SemaphoreType.DMA((N_HOPS,)),
            pltpu.SemaphoreType.DMA((N_HOPS,)),
            pltpu.SemaphoreType.DMA((N_HOPS,)),
            pltpu.SemaphoreType.DMA((16,)),
            pltpu.SemaphoreType.DMA((16,)),
            pltpu.SemaphoreType.DMA((16,)),
            pltpu.SemaphoreType.DMA((16,)),
            pltpu.SemaphoreType.DMA((16,)),
            pltpu.SemaphoreType.DMA((16,)),
            pltpu.SemaphoreType.DMA((16,)),
            pltpu.SemaphoreType.DMA((16,)),
        ],
        compiler_params=pltpu.CompilerParams(collective_id=0),
    )(ids_p, mask_p, E)


def kernel(ids, E):
    v_local = E.shape[0]
    my_x = lax.axis_index("x")
    my_y = lax.axis_index("y")
    my_z = lax.axis_index("z")
    my_r = jnp.where(my_x == 0, my_y, 7 - my_y)
    ids_p = lax.dynamic_slice_in_dim(ids, my_r * P, P)
    local = ids_p - my_z * v_local
    in_range = (local >= 0) & (local < v_local)
    safe_ids = jnp.where(in_range, local, 0)
    mask = in_range[:, None].astype(jnp.float32)
    return _vembed_allreduce(safe_ids, mask, E)


# device time: 80078 ns/iter; 2.7110x vs baseline; 1.0697x over previous
import jax
import jax.numpy as jnp
from jax import lax
from jax.experimental import pallas as pl
from jax.experimental.pallas import tpu as pltpu

N_Z = 4
N_XY = 8
T = 2048
D = 1024
P = T // N_XY
HALF = P // 2
CH = HALF // N_Z
N_HOPS = 2 * (N_Z - 1)


def _xy_coords(t):
    x = jnp.where(t < 4, 0, 1)
    y = jnp.where(t < 4, t, 7 - t)
    return x, y


def _vembed_allreduce(ids_p, mask_p, E):
    def body(ids_ref, mask_ref, e_ref, out_ref, gat_ref, red_ref,
             comm_a, comm_b, gsem,
             send_a, recv_a, send_b, recv_b,
             send_ra, recv_ra, send_rb, recv_rb,
             send_la, recv_la, send_lb, recv_lb):
        my_x = lax.axis_index("x")
        my_y = lax.axis_index("y")
        my_z = lax.axis_index("z")
        z_right = (my_x, my_y, (my_z + 1) % N_Z)
        z_left = (my_x, my_y, (my_z - 1) % N_Z)

        my_r = jnp.where(my_x == 0, my_y, 7 - my_y)
        rx, ry = _xy_coords((my_r + 1) % N_XY)
        lx, ly = _xy_coords((my_r + N_XY - 1) % N_XY)
        xy_right = (rx, ry, my_z)
        xy_left = (lx, ly, my_z)

        def issue(i, c):
            pltpu.make_async_copy(
                e_ref.at[pl.ds(ids_ref[i], 1)], gat_ref.at[pl.ds(i, 1)], gsem
            ).start()
            return c

        lax.fori_loop(0, P, issue, 0)

        barrier_sem = pltpu.get_barrier_semaphore()
        for nbr in (z_left, z_right, xy_left, xy_right):
            pl.semaphore_signal(
                barrier_sem,
                inc=1,
                device_id=nbr,
                device_id_type=pl.DeviceIdType.MESH,
            )
        pl.semaphore_wait(barrier_sem, 4)

        def drain(i, c):
            pltpu.make_async_copy(
                e_ref.at[pl.ds(0, 1)], gat_ref.at[pl.ds(0, 1)], gsem
            ).wait()
            return c

        lax.fori_loop(0, P, drain, 0)

        red_ref[:, :] = jnp.where(mask_ref[:, :] != 0, gat_ref[:, :], 0.0)

        def a_base(k):
            return ((my_z - k) % N_Z) * CH

        def b_base(k):
            return HALF + ((my_z + k) % N_Z) * CH

        def hop_rdmas(h, src_a, src_b):
            rdma_a = pltpu.make_async_remote_copy(
                src_ref=src_a,
                dst_ref=comm_a.at[h],
                send_sem=send_a.at[h],
                recv_sem=recv_a.at[h],
                device_id=z_right,
                device_id_type=pl.DeviceIdType.MESH,
            )
            rdma_b = pltpu.make_async_remote_copy(
                src_ref=src_b,
                dst_ref=comm_b.at[h],
                send_sem=send_b.at[h],
                recv_sem=recv_b.at[h],
                device_id=z_left,
                device_id_type=pl.DeviceIdType.MESH,
            )
            return rdma_a, rdma_b

        for h in range(N_Z - 1):
            rdma_a, rdma_b = hop_rdmas(
                h,
                red_ref.at[pl.ds(a_base(h), CH)],
                red_ref.at[pl.ds(b_base(h), CH)],
            )
            rdma_a.start()
            rdma_b.start()
            rdma_a.wait()
            rdma_b.wait()
            ra = a_base(h + 1)
            rb = b_base(h + 1)
            red_ref[pl.ds(ra, CH), :] = red_ref[pl.ds(ra, CH), :] + comm_a[h, :, :]
            red_ref[pl.ds(rb, CH), :] = red_ref[pl.ds(rb, CH), :] + comm_b[h, :, :]

        def lane_ab(L):
            if L == 0:
                return (my_z + 1) % N_Z, (my_z - 1) % N_Z
            return (my_z - (L - 1)) % N_Z, (my_z + (L - 1)) % N_Z

        def lane_hop_rdmas(L, t):
            aL, bL = lane_ab(L)
            i = L * 4 + t
            qr = ((my_r - t) % N_XY) * P
            ql = ((my_r + t) % N_XY) * P

            def mk(rows, send, recv, dev):
                return pltpu.make_async_remote_copy(
                    src_ref=out_ref.at[pl.ds(rows, CH)],
                    dst_ref=out_ref.at[pl.ds(rows, CH)],
                    send_sem=send.at[i],
                    recv_sem=recv.at[i],
                    device_id=dev,
                    device_id_type=pl.DeviceIdType.MESH,
                )

            ra = mk(qr + aL * CH, send_ra, recv_ra, xy_right)
            rb = mk(qr + HALF + bL * CH, send_rb, recv_rb, xy_right)
            la = mk(ql + aL * CH, send_la, recv_la, xy_left)
            lb = mk(ql + HALF + bL * CH, send_lb, recv_lb, xy_left)
            return ra, rb, la, lb

        def start_lane_hop(L, t):
            ra, rb, la, lb = lane_hop_rdmas(L, t)
            ra.start()
            lb.start()
            if t < N_XY // 2 - 1:
                rb.start()
                la.start()

        def wait_lane_hop(L, t):
            ra, rb, la, lb = lane_hop_rdmas(L, t)
            ra.wait()
            lb.wait()
            if t < N_XY // 2 - 1:
                rb.wait()
                la.wait()

        a0, b0 = lane_ab(0)
        my_base = my_r * P
        out_ref[pl.ds(my_base + a0 * CH, CH), :] = red_ref[pl.ds(a0 * CH, CH), :]
        out_ref[pl.ds(my_base + HALF + b0 * CH, CH), :] = (
            red_ref[pl.ds(HALF + b0 * CH, CH), :]
        )
        start_lane_hop(0, 0)

        for e in range(1, N_Z + N_XY // 2 - 1):
            if e <= N_Z - 1:
                g = e - 1
                h = (N_Z - 1) + g
                if g == 0:
                    src_a = red_ref.at[pl.ds(a_base(-1), CH)]
                    src_b = red_ref.at[pl.ds(b_base(-1), CH)]
                else:
                    src_a = comm_a.at[h - 1]
                    src_b = comm_b.at[h - 1]
                rdma_a, rdma_b = hop_rdmas(h, src_a, src_b)
                rdma_a.start()
                rdma_b.start()
                rdma_a.wait()
                rdma_b.wait()
                out_ref[pl.ds(my_base + a_base(g), CH), :] = comm_a[h, :, :]
                out_ref[pl.ds(my_base + b_base(g), CH), :] = comm_b[h, :, :]
                start_lane_hop(e, 0)
            for L in range(N_Z):
                t = e - L
                if 1 <= t <= N_XY // 2 - 1:
                    wait_lane_hop(L, t - 1)
                    start_lane_hop(L, t)
        for L in range(N_Z):
            wait_lane_hop(L, N_XY // 2 - 1)

    return pl.pallas_call(
        body,
        out_shape=jax.ShapeDtypeStruct((T, D), jnp.float32),
        in_specs=[
            pl.BlockSpec(memory_space=pltpu.SMEM),
            pl.BlockSpec(memory_space=pltpu.VMEM),
            pl.BlockSpec(memory_space=pl.ANY),
        ],
        out_specs=pl.BlockSpec(memory_space=pltpu.VMEM),
        scratch_shapes=[
            pltpu.VMEM((P, D), jnp.float32),
            pltpu.VMEM((P, D), jnp.float32),
            pltpu.VMEM((N_HOPS, CH, D), jnp.float32),
            pltpu.VMEM((N_HOPS, CH, D), jnp.float32),
            pltpu.SemaphoreType.DMA,
            pltpu.SemaphoreType.DMA((N_HOPS,)),
            pltpu.SemaphoreType.DMA((N_HOPS,)),
            pltpu.SemaphoreType.DMA((N_HOPS,)),
            pltpu.SemaphoreType.DMA((N_HOPS,)),
            pltpu.SemaphoreType.DMA((16,)),
            pltpu.SemaphoreType.DMA((16,)),
            pltpu.SemaphoreType.DMA((16,)),
            pltpu.SemaphoreType.DMA((16,)),
            pltpu.SemaphoreType.DMA((16,)),
            pltpu.SemaphoreType.DMA((16,)),
            pltpu.SemaphoreType.DMA((16,)),
            pltpu.SemaphoreType.DMA((16,)),
        ],
        compiler_params=pltpu.CompilerParams(collective_id=0),
    )(ids_p, mask_p, E)


def kernel(ids, E):
    v_local = E.shape[0]
    my_x = lax.axis_index("x")
    my_y = lax.axis_index("y")
    my_z = lax.axis_index("z")
    my_r = jnp.where(my_x == 0, my_y, 7 - my_y)
    ids_p = lax.dynamic_slice_in_dim(ids, my_r * P, P)
    local = ids_p - my_z * v_local
    in_range = (local >= 0) & (local < v_local)
    safe_ids = jnp.where(in_range, local, 0)
    mask = in_range[:, None].astype(jnp.float32)
    return _vembed_allreduce(safe_ids, mask, E)


# device time: 79129 ns/iter; 2.7436x vs baseline; 1.0120x over previous
import jax
import jax.numpy as jnp
from jax import lax
from jax.experimental import pallas as pl
from jax.experimental.pallas import tpu as pltpu

N_Z = 4
N_XY = 8
T = 2048
D = 1024
P = T // N_XY
HALF = P // 2
CH = HALF // N_Z
N_HOPS = 2 * (N_Z - 1)


def _xy_coords(t):
    x = jnp.where(t < 4, 0, 1)
    y = jnp.where(t < 4, t, 7 - t)
    return x, y


def _vembed_allreduce(ids_p, mask_p, E):
    def body(ids_ref, mask_ref, e_ref, out_ref, gat_ref, red_ref,
             comm_a, comm_b, gsem0, gsem1,
             send_a, recv_a, send_b, recv_b,
             send_ra, recv_ra, send_rb, recv_rb,
             send_la, recv_la, send_lb, recv_lb):
        my_x = lax.axis_index("x")
        my_y = lax.axis_index("y")
        my_z = lax.axis_index("z")
        z_right = (my_x, my_y, (my_z + 1) % N_Z)
        z_left = (my_x, my_y, (my_z - 1) % N_Z)

        my_r = jnp.where(my_x == 0, my_y, 7 - my_y)
        rx, ry = _xy_coords((my_r + 1) % N_XY)
        lx, ly = _xy_coords((my_r + N_XY - 1) % N_XY)
        xy_right = (rx, ry, my_z)
        xy_left = (lx, ly, my_z)

        def issue_rows(base, sem):
            def f(i, c):
                pltpu.make_async_copy(
                    e_ref.at[pl.ds(ids_ref[base + i], 1)],
                    gat_ref.at[pl.ds(base + i, 1)],
                    sem,
                ).start()
                return c

            lax.fori_loop(0, CH, f, 0)

        rest_blocks = []
        for k in range(N_Z):
            ba = ((my_z + k) % N_Z) * CH
            bb = HALF + ((my_z + k) % N_Z) * CH
            if k == 0:
                issue_rows(ba, gsem0)
                issue_rows(bb, gsem0)
            else:
                rest_blocks += [ba, bb]
        for b in rest_blocks:
            issue_rows(b, gsem1)

        barrier_sem = pltpu.get_barrier_semaphore()
        for nbr in (z_left, z_right, xy_left, xy_right):
            pl.semaphore_signal(
                barrier_sem,
                inc=1,
                device_id=nbr,
                device_id_type=pl.DeviceIdType.MESH,
            )
        pl.semaphore_wait(barrier_sem, 4)

        def drain(sem, n):
            def w(i, c):
                pltpu.make_async_copy(
                    e_ref.at[pl.ds(0, 1)], gat_ref.at[pl.ds(0, 1)], sem
                ).wait()
                return c

            lax.fori_loop(0, n, w, 0)

        def mask_chunk(base):
            red_ref[pl.ds(base, CH), :] = jnp.where(
                mask_ref[pl.ds(base, CH), :] != 0,
                gat_ref[pl.ds(base, CH), :],
                0.0,
            )

        drain(gsem0, 2 * CH)
        mask_chunk(my_z * CH)
        mask_chunk(HALF + my_z * CH)

        def a_base(k):
            return ((my_z - k) % N_Z) * CH

        def b_base(k):
            return HALF + ((my_z + k) % N_Z) * CH

        def hop_rdmas(h, src_a, src_b):
            rdma_a = pltpu.make_async_remote_copy(
                src_ref=src_a,
                dst_ref=comm_a.at[h],
                send_sem=send_a.at[h],
                recv_sem=recv_a.at[h],
                device_id=z_right,
                device_id_type=pl.DeviceIdType.MESH,
            )
            rdma_b = pltpu.make_async_remote_copy(
                src_ref=src_b,
                dst_ref=comm_b.at[h],
                send_sem=send_b.at[h],
                recv_sem=recv_b.at[h],
                device_id=z_left,
                device_id_type=pl.DeviceIdType.MESH,
            )
            return rdma_a, rdma_b

        for h in range(N_Z - 1):
            rdma_a, rdma_b = hop_rdmas(
                h,
                red_ref.at[pl.ds(a_base(h), CH)],
                red_ref.at[pl.ds(b_base(h), CH)],
            )
            rdma_a.start()
            rdma_b.start()
            if h == 0:
                drain(gsem1, 6 * CH)
                for b in rest_blocks:
                    mask_chunk(b)
            rdma_a.wait()
            rdma_b.wait()
            ra = a_base(h + 1)
            rb = b_base(h + 1)
            red_ref[pl.ds(ra, CH), :] = red_ref[pl.ds(ra, CH), :] + comm_a[h, :, :]
            red_ref[pl.ds(rb, CH), :] = red_ref[pl.ds(rb, CH), :] + comm_b[h, :, :]

        def lane_ab(L):
            if L == 0:
                return (my_z + 1) % N_Z, (my_z - 1) % N_Z
            return (my_z - (L - 1)) % N_Z, (my_z + (L - 1)) % N_Z

        def lane_hop_rdmas(L, t):
            aL, bL = lane_ab(L)
            i = L * 4 + t
            qr = ((my_r - t) % N_XY) * P
            ql = ((my_r + t) % N_XY) * P

            def mk(rows, send, recv, dev):
                return pltpu.make_async_remote_copy(
                    src_ref=out_ref.at[pl.ds(rows, CH)],
                    dst_ref=out_ref.at[pl.ds(rows, CH)],
                    send_sem=send.at[i],
                    recv_sem=recv.at[i],
                    device_id=dev,
                    device_id_type=pl.DeviceIdType.MESH,
                )

            ra = mk(qr + aL * CH, send_ra, recv_ra, xy_right)
            rb = mk(qr + HALF + bL * CH, send_rb, recv_rb, xy_right)
            la = mk(ql + aL * CH, send_la, recv_la, xy_left)
            lb = mk(ql + HALF + bL * CH, send_lb, recv_lb, xy_left)
            return ra, rb, la, lb

        def start_lane_hop(L, t):
            ra, rb, la, lb = lane_hop_rdmas(L, t)
            ra.start()
            lb.start()
            if t < N_XY // 2 - 1:
                rb.start()
                la.start()

        def wait_lane_hop(L, t):
            ra, rb, la, lb = lane_hop_rdmas(L, t)
            ra.wait()
            lb.wait()
            if t < N_XY // 2 - 1:
                rb.wait()
                la.wait()

        a0, b0 = lane_ab(0)
        my_base = my_r * P
        out_ref[pl.ds(my_base + a0 * CH, CH), :] = red_ref[pl.ds(a0 * CH, CH), :]
        out_ref[pl.ds(my_base + HALF + b0 * CH, CH), :] = (
            red_ref[pl.ds(HALF + b0 * CH, CH), :]
        )
        start_lane_hop(0, 0)

        for e in range(1, N_Z + N_XY // 2 - 1):
            if e <= N_Z - 1:
                g = e - 1
                h = (N_Z - 1) + g
                if g == 0:
                    src_a = red_ref.at[pl.ds(a_base(-1), CH)]
                    src_b = red_ref.at[pl.ds(b_base(-1), CH)]
                else:
                    src_a = comm_a.at[h - 1]
                    src_b = comm_b.at[h - 1]
                rdma_a, rdma_b = hop_rdmas(h, src_a, src_b)
                rdma_a.start()
                rdma_b.start()
                rdma_a.wait()
                rdma_b.wait()
                out_ref[pl.ds(my_base + a_base(g), CH), :] = comm_a[h, :, :]
                out_ref[pl.ds(my_base + b_base(g), CH), :] = comm_b[h, :, :]
                start_lane_hop(e, 0)
            for L in range(N_Z):
                t = e - L
                if 1 <= t <= N_XY // 2 - 1:
                    wait_lane_hop(L, t - 1)
                    start_lane_hop(L, t)
        for L in range(N_Z):
            wait_lane_hop(L, N_XY // 2 - 1)

    return pl.pallas_call(
        body,
        out_shape=jax.ShapeDtypeStruct((T, D), jnp.float32),
        in_specs=[
            pl.BlockSpec(memory_space=pltpu.SMEM),
            pl.BlockSpec(memory_space=pltpu.VMEM),
            pl.BlockSpec(memory_space=pl.ANY),
        ],
        out_specs=pl.BlockSpec(memory_space=pltpu.VMEM),
        scratch_shapes=[
            pltpu.VMEM((P, D), jnp.float32),
            pltpu.VMEM((P, D), jnp.float32),
            pltpu.VMEM((N_HOPS, CH, D), jnp.float32),
            pltpu.VMEM((N_HOPS, CH, D), jnp.float32),
            pltpu.SemaphoreType.DMA,
            pltpu.SemaphoreType.DMA,
            pltpu.SemaphoreType.DMA((N_HOPS,)),
            pltpu.SemaphoreType.DMA((N_HOPS,)),
            pltpu.SemaphoreType.DMA((N_HOPS,)),
            pltpu.SemaphoreType.DMA((N_HOPS,)),
            pltpu.SemaphoreType.DMA((16,)),
            pltpu.SemaphoreType.DMA((16,)),
            pltpu.SemaphoreType.DMA((16,)),
            pltpu.SemaphoreType.DMA((16,)),
            pltpu.SemaphoreType.DMA((16,)),
            pltpu.SemaphoreType.DMA((16,)),
            pltpu.SemaphoreType.DMA((16,)),
            pltpu.SemaphoreType.DMA((16,)),
        ],
        compiler_params=pltpu.CompilerParams(collective_id=0),
    )(ids_p, mask_p, E)


def kernel(ids, E):
    v_local = E.shape[0]
    my_x = lax.axis_index("x")
    my_y = lax.axis_index("y")
    my_z = lax.axis_index("z")
    my_r = jnp.where(my_x == 0, my_y, 7 - my_y)
    ids_p = lax.dynamic_slice_in_dim(ids, my_r * P, P)
    local = ids_p - my_z * v_local
    in_range = (local >= 0) & (local < v_local)
    safe_ids = jnp.where(in_range, local, 0)
    mask = in_range[:, None].astype(jnp.float32)
    return _vembed_allreduce(safe_ids, mask, E)


# device time: 76816 ns/iter; 2.8262x vs baseline; 1.0301x over previous
import jax
import jax.numpy as jnp
from jax import lax
from jax.experimental import pallas as pl
from jax.experimental.pallas import tpu as pltpu

N_Z = 4
N_XY = 8
T = 2048
D = 1024
P = T // N_XY
HALF = P // 2
CH = HALF // N_Z
N_HOPS = 2 * (N_Z - 1)


def _xy_coords(t):
    x = jnp.where(t < 4, 0, 1)
    y = jnp.where(t < 4, t, 7 - t)
    return x, y


def _vembed_allreduce(ids_p, mask_p, E):
    def body(ids_ref, mask_ref, e_ref, out_ref, gat_ref, red_ref,
             comm_a, comm_b, gsem0, gsem1,
             send_a, recv_a, send_b, recv_b,
             send_ra, recv_ra, send_rb, recv_rb,
             send_la, recv_la, send_lb, recv_lb):
        my_x = lax.axis_index("x")
        my_y = lax.axis_index("y")
        my_z = lax.axis_index("z")
        z_right = (my_x, my_y, (my_z + 1) % N_Z)
        z_left = (my_x, my_y, (my_z - 1) % N_Z)

        my_r = jnp.where(my_x == 0, my_y, 7 - my_y)
        rx, ry = _xy_coords((my_r + 1) % N_XY)
        lx, ly = _xy_coords((my_r + N_XY - 1) % N_XY)
        xy_right = (rx, ry, my_z)
        xy_left = (lx, ly, my_z)

        def issue_rows(base, sem):
            def f(i, c):
                pltpu.make_async_copy(
                    e_ref.at[pl.ds(ids_ref[base + i], 1)],
                    gat_ref.at[pl.ds(base + i, 1)],
                    sem,
                ).start()
                return c

            lax.fori_loop(0, CH, f, 0, unroll=8)

        rest_blocks = []
        for k in range(N_Z):
            ba = ((my_z + k) % N_Z) * CH
            bb = HALF + ((my_z + k) % N_Z) * CH
            if k == 0:
                issue_rows(ba, gsem0)
                issue_rows(bb, gsem0)
            else:
                rest_blocks += [ba, bb]
        for b in rest_blocks:
            issue_rows(b, gsem1)

        barrier_sem = pltpu.get_barrier_semaphore()
        for nbr in (z_left, z_right, xy_left, xy_right):
            pl.semaphore_signal(
                barrier_sem,
                inc=1,
                device_id=nbr,
                device_id_type=pl.DeviceIdType.MESH,
            )
        pl.semaphore_wait(barrier_sem, 4)

        def drain(sem, n):
            def w(i, c):
                pltpu.make_async_copy(
                    e_ref.at[pl.ds(0, 1)], gat_ref.at[pl.ds(0, 1)], sem
                ).wait()
                return c

            lax.fori_loop(0, n, w, 0, unroll=8)

        def mask_chunk(base):
            red_ref[pl.ds(base, CH), :] = jnp.where(
                mask_ref[pl.ds(base, CH), :] != 0,
                gat_ref[pl.ds(base, CH), :],
                0.0,
            )

        drain(gsem0, 2 * CH)
        mask_chunk(my_z * CH)
        mask_chunk(HALF + my_z * CH)

        def a_base(k):
            return ((my_z - k) % N_Z) * CH

        def b_base(k):
            return HALF + ((my_z + k) % N_Z) * CH

        def hop_rdmas(h, src_a, src_b):
            rdma_a = pltpu.make_async_remote_copy(
                src_ref=src_a,
                dst_ref=comm_a.at[h],
                send_sem=send_a.at[h],
                recv_sem=recv_a.at[h],
                device_id=z_right,
                device_id_type=pl.DeviceIdType.MESH,
            )
            rdma_b = pltpu.make_async_remote_copy(
                src_ref=src_b,
                dst_ref=comm_b.at[h],
                send_sem=send_b.at[h],
                recv_sem=recv_b.at[h],
                device_id=z_left,
                device_id_type=pl.DeviceIdType.MESH,
            )
            return rdma_a, rdma_b

        for h in range(N_Z - 1):
            rdma_a, rdma_b = hop_rdmas(
                h,
                red_ref.at[pl.ds(a_base(h), CH)],
                red_ref.at[pl.ds(b_base(h), CH)],
            )
            rdma_a.start()
            rdma_b.start()
            if h == 0:
                drain(gsem1, 6 * CH)
                for b in rest_blocks:
                    mask_chunk(b)
            rdma_a.wait()
            rdma_b.wait()
            ra = a_base(h + 1)
            rb = b_base(h + 1)
            red_ref[pl.ds(ra, CH), :] = red_ref[pl.ds(ra, CH), :] + comm_a[h, :, :]
            red_ref[pl.ds(rb, CH), :] = red_ref[pl.ds(rb, CH), :] + comm_b[h, :, :]

        def lane_ab(L):
            if L == 0:
                return (my_z + 1) % N_Z, (my_z - 1) % N_Z
            return (my_z - (L - 1)) % N_Z, (my_z + (L - 1)) % N_Z

        def lane_hop_rdmas(L, t):
            aL, bL = lane_ab(L)
            i = L * 4 + t
            qr = ((my_r - t) % N_XY) * P
            ql = ((my_r + t) % N_XY) * P

            def mk(rows, send, recv, dev):
                return pltpu.make_async_remote_copy(
                    src_ref=out_ref.at[pl.ds(rows, CH)],
                    dst_ref=out_ref.at[pl.ds(rows, CH)],
                    send_sem=send.at[i],
                    recv_sem=recv.at[i],
                    device_id=dev,
                    device_id_type=pl.DeviceIdType.MESH,
                )

            ra = mk(qr + aL * CH, send_ra, recv_ra, xy_right)
            rb = mk(qr + HALF + bL * CH, send_rb, recv_rb, xy_right)
            la = mk(ql + aL * CH, send_la, recv_la, xy_left)
            lb = mk(ql + HALF + bL * CH, send_lb, recv_lb, xy_left)
            return ra, rb, la, lb

        def start_lane_hop(L, t):
            ra, rb, la, lb = lane_hop_rdmas(L, t)
            ra.start()
            lb.start()
            if t < N_XY // 2 - 1:
                rb.start()
                la.start()

        def wait_lane_hop(L, t):
            ra, rb, la, lb = lane_hop_rdmas(L, t)
            ra.wait()
            lb.wait()
            if t < N_XY // 2 - 1:
                rb.wait()
                la.wait()

        a0, b0 = lane_ab(0)
        my_base = my_r * P
        out_ref[pl.ds(my_base + a0 * CH, CH), :] = red_ref[pl.ds(a0 * CH, CH), :]
        out_ref[pl.ds(my_base + HALF + b0 * CH, CH), :] = (
            red_ref[pl.ds(HALF + b0 * CH, CH), :]
        )
        start_lane_hop(0, 0)

        for e in range(1, N_Z + N_XY // 2 - 1):
            if e <= N_Z - 1:
                g = e - 1
                h = (N_Z - 1) + g
                if g == 0:
                    src_a = red_ref.at[pl.ds(a_base(-1), CH)]
                    src_b = red_ref.at[pl.ds(b_base(-1), CH)]
                else:
                    src_a = comm_a.at[h - 1]
                    src_b = comm_b.at[h - 1]
                rdma_a, rdma_b = hop_rdmas(h, src_a, src_b)
                rdma_a.start()
                rdma_b.start()
                rdma_a.wait()
                rdma_b.wait()
                out_ref[pl.ds(my_base + a_base(g), CH), :] = comm_a[h, :, :]
                out_ref[pl.ds(my_base + b_base(g), CH), :] = comm_b[h, :, :]
                start_lane_hop(e, 0)
            for L in range(N_Z):
                t = e - L
                if 1 <= t <= N_XY // 2 - 1:
                    wait_lane_hop(L, t - 1)
                    start_lane_hop(L, t)
        for L in range(N_Z):
            wait_lane_hop(L, N_XY // 2 - 1)

    return pl.pallas_call(
        body,
        out_shape=jax.ShapeDtypeStruct((T, D), jnp.float32),
        in_specs=[
            pl.BlockSpec(memory_space=pltpu.SMEM),
            pl.BlockSpec(memory_space=pltpu.VMEM),
            pl.BlockSpec(memory_space=pl.ANY),
        ],
        out_specs=pl.BlockSpec(memory_space=pltpu.VMEM),
        scratch_shapes=[
            pltpu.VMEM((P, D), jnp.float32),
            pltpu.VMEM((P, D), jnp.float32),
            pltpu.VMEM((N_HOPS, CH, D), jnp.float32),
            pltpu.VMEM((N_HOPS, CH, D), jnp.float32),
            pltpu.SemaphoreType.DMA,
            pltpu.SemaphoreType.DMA,
            pltpu.SemaphoreType.DMA((N_HOPS,)),
            pltpu.SemaphoreType.DMA((N_HOPS,)),
            pltpu.SemaphoreType.DMA((N_HOPS,)),
            pltpu.SemaphoreType.DMA((N_HOPS,)),
            pltpu.SemaphoreType.DMA((16,)),
            pltpu.SemaphoreType.DMA((16,)),
            pltpu.SemaphoreType.DMA((16,)),
            pltpu.SemaphoreType.DMA((16,)),
            pltpu.SemaphoreType.DMA((16,)),
            pltpu.SemaphoreType.DMA((16,)),
            pltpu.SemaphoreType.DMA((16,)),
            pltpu.SemaphoreType.DMA((16,)),
        ],
        compiler_params=pltpu.CompilerParams(collective_id=0),
    )(ids_p, mask_p, E)


def kernel(ids, E):
    v_local = E.shape[0]
    my_x = lax.axis_index("x")
    my_y = lax.axis_index("y")
    my_z = lax.axis_index("z")
    my_r = jnp.where(my_x == 0, my_y, 7 - my_y)
    ids_p = lax.dynamic_slice_in_dim(ids, my_r * P, P)
    local = ids_p - my_z * v_local
    in_range = (local >= 0) & (local < v_local)
    safe_ids = jnp.where(in_range, local, 0)
    mask = in_range[:, None].astype(jnp.float32)
    return _vembed_allreduce(safe_ids, mask, E)
